# baseline (speedup 1.0000x reference)
import functools

import jax
import jax.numpy as jnp
from jax import lax
from jax.experimental import pallas as pl
from jax.experimental.pallas import tpu as pltpu
from jax.experimental.pallas import tpu_sc as plsc

_NUM_REL = 237
_D = 128
_N = 10000
_E = 320000

_TPAD = 512
_KP = _TPAD // _D
_NQ = 4
_QROWS = _N // _NQ
_PLANE = _QROWS * _D
_NHQ = _QROWS * _TPAD
_TRASH = _NHQ
_HBUF = _NHQ + _TPAD

_NC = 2
_NS = 16
_SLICE = _NHQ // _NS
_ZCH = 8000

_BATCH = 2048
_NBATCH = 10
_EPT = _E // _NS
_TAIL = _EPT - (_NBATCH - 1) * _BATCH
_TGRP = _TAIL // 16


def _sc_hist_body(et_hbm, dst_hbm, out_hbm, hist,
                  ib0, ib1, ones_v, zb, et0, dst0, et1, dst1,
                  lsem, ssem, zsem):
  c = lax.axis_index("c")
  s = lax.axis_index("s")
  ebufs = ((et0, dst0, ib0), (et1, dst1, ib1))


  def _init_z(i, _):
    zb[pl.ds(i * 16, 16)] = jnp.zeros((16,), jnp.float32)
    return ()
  lax.fori_loop(0, _ZCH // 16, _init_z, ())

  def _init_o(i, _):
    ones_v[pl.ds(i * 16, 16)] = jnp.ones((16,), jnp.float32)
    return ()
  lax.fori_loop(0, _BATCH // 16, _init_o, ())

  for p in range(2):
    q = 2 * p + c
    lo = q * _QROWS

    zds = [
        pltpu.async_copy(
            zb, hist.at[pl.ds(s * _SLICE + k * _ZCH, _ZCH)], zsem)
        for k in range(_SLICE // _ZCH)
    ]

    def _start_loads(b):
      n = _BATCH if b < _NBATCH - 1 else _TAIL
      ebase = s * _EPT + b * _BATCH
      et_v, dst_v, _ = ebufs[b % 2]
      return (
          pltpu.async_copy(et_hbm.at[pl.ds(ebase, n)],
                           et_v.at[pl.ds(0, n)], lsem),
          pltpu.async_copy(dst_hbm.at[pl.ds(ebase, n)],
                           dst_v.at[pl.ds(0, n)], lsem),
      )

    def _group(g, et_v, dst_v, ib):
      off = g * 16
      et = et_v[pl.ds(off, 16)]
      rel = dst_v[pl.ds(off, 16)] - lo
      inq = (rel >= 0) & (rel < _QROWS)
      idx = jnp.where(
          inq, (et >> 7) * _PLANE + rel * _D, _TRASH) + (et & 127)
      ib[pl.ds(off, 16)] = idx

    def _compute(b):
      et_v, dst_v, ib = ebufs[b % 2]
      if b < _NBATCH - 1:
        def _rows(r, _):
          for c8 in range(8):
            _group(r * 8 + c8, et_v, dst_v, ib)
          return ()
        lax.fori_loop(0, 16, _rows, ())
      else:
        def _rows_t(r, _):
          for c8 in range(8):
            _group(r * 8 + c8, et_v, dst_v, ib)
          return ()
        lax.fori_loop(0, _TGRP // 8, _rows_t, ())
        for g in range(_TGRP - _TGRP % 8, _TGRP):
          _group(g, et_v, dst_v, ib)
        tv = _TRASH + lax.iota(jnp.int32, 16)
        def _tf(g, _):
          ib[pl.ds(g * 16, 16)] = tv
          return ()
        lax.fori_loop(_TGRP, _BATCH // 16, _tf, ())

    pend = {0: _start_loads(0), 1: _start_loads(1)}
    for d in pend.pop(0):
      d.wait()
    _compute(0)
    for d in zds:
      d.wait()
    plsc.subcore_barrier()

    pending_scatter = [
        pltpu.async_copy(ones_v, hist.at[ebufs[0][2]], ssem, add=True)]
    pend[2] = _start_loads(2)
    for b in range(1, _NBATCH):
      et_v, dst_v, ib = ebufs[b % 2]
      for d in pend.pop(b):
        d.wait()
      _compute(b)
      for d in pending_scatter:
        d.wait()
      pending_scatter = [
          pltpu.async_copy(ones_v, hist.at[ib], ssem, add=True)]
      if b + 2 < _NBATCH:
        pend[b + 2] = _start_loads(b + 2)
    for d in pending_scatter:
      d.wait()
    plsc.subcore_barrier()

    kpl = s // 4
    j0 = (s % 4) * _SLICE
    pltpu.sync_copy(
        hist.at[pl.ds(s * _SLICE, _SLICE)],
        out_hbm.at[pl.ds(kpl * (_N * _D) + q * _PLANE + j0, _SLICE)])


_sc_hist = functools.partial(
    pl.kernel,
    out_type=jax.ShapeDtypeStruct((_NQ * _NHQ,), jnp.float32),
    mesh=plsc.VectorSubcoreMesh(
        core_axis_name="c", subcore_axis_name="s",
        num_cores=_NC, num_subcores=_NS),
    scratch_types=[
        pltpu.VMEM_SHARED((_HBUF,), jnp.float32),
        pltpu.VMEM((_BATCH,), jnp.int32),
        pltpu.VMEM((_BATCH,), jnp.int32),
        pltpu.VMEM((_BATCH,), jnp.float32),
        pltpu.VMEM((_ZCH,), jnp.float32),
        pltpu.VMEM((_BATCH,), jnp.int32),
        pltpu.VMEM((_BATCH,), jnp.int32),
        pltpu.VMEM((_BATCH,), jnp.int32),
        pltpu.VMEM((_BATCH,), jnp.int32),
        pltpu.SemaphoreType.DMA,
        pltpu.SemaphoreType.DMA,
        pltpu.SemaphoreType.DMA,
    ],
)(_sc_hist_body)


def _mm_body(h_ref, t_ref, o_ref):
  acc = jnp.zeros((h_ref.shape[1], _D), jnp.float32)
  deg = jnp.zeros((h_ref.shape[1], 1), jnp.float32)
  for k in range(_KP):
    hk = h_ref[k]
    acc += jnp.dot(hk, t_ref[k], preferred_element_type=jnp.float32)
    deg += jnp.sum(hk, axis=1, keepdims=True)
  o_ref[...] = acc / jnp.maximum(deg, 1.0)


def _tc_combine(hist3, table3):
  blk = 1000
  return pl.pallas_call(
      _mm_body,
      grid=(_N // blk,),
      in_specs=[
          pl.BlockSpec((_KP, blk, _D), lambda i: (0, i, 0)),
          pl.BlockSpec((_KP, _D, _D), lambda i: (0, 0, 0)),
      ],
      out_specs=pl.BlockSpec((blk, _D), lambda i: (i, 0)),
      out_shape=jax.ShapeDtypeStruct((_N, _D), jnp.float32),
  )(hist3, table3)


@jax.jit
def kernel(rel_head_emb, rel_tail_emb, etypes, dst):
  table = jnp.zeros((_TPAD, _D), jnp.float32)
  table = lax.dynamic_update_slice(table, rel_head_emb, (0, 0))
  table = lax.dynamic_update_slice(table, rel_tail_emb, (_NUM_REL, 0))
  table3 = table.reshape(_KP, _D, _D)

  hist3 = _sc_hist(etypes, dst).reshape(_KP, _N, _D)
  return _tc_combine(hist3, table3)


# speedup vs baseline: 1.0366x; 1.0366x over previous
import functools

import jax
import jax.numpy as jnp
from jax import lax
from jax.experimental import pallas as pl
from jax.experimental.pallas import tpu as pltpu
from jax.experimental.pallas import tpu_sc as plsc

_NUM_REL = 237
_D = 128
_N = 10000
_E = 320000

_TPAD = 512
_KP = _TPAD // _D
_NQ = 4
_QROWS = _N // _NQ
_PLANE = _QROWS * _D
_NHQ = _QROWS * _TPAD
_TRASH = _NHQ
_HBUF = _NHQ + _TPAD

_NC = 2
_NS = 16
_SLICE = _NHQ // _NS
_ZCH = 8000

_BATCH = 2048
_NBATCH = 10
_EPT = _E // _NS
_TAIL = _EPT - (_NBATCH - 1) * _BATCH
_TGRP = _TAIL // 16


_RCH = 16000
_NRCH = _SLICE // _RCH


def _sc_hist_body(et_hbm, dst_hbm, out_hbm, hist,
                  ib0, ib1, ones_v, zb, et0, dst0, et1, dst1,
                  lsem, ssem, zsem, rsem):
  c = lax.axis_index("c")
  s = lax.axis_index("s")
  ebufs = ((et0, dst0, ib0), (et1, dst1, ib1))


  def _init_z(i, _):
    zb[pl.ds(i * 16, 16)] = jnp.zeros((16,), jnp.float32)
    return ()
  lax.fori_loop(0, _ZCH // 16, _init_z, ())

  def _init_o(i, _):
    ones_v[pl.ds(i * 16, 16)] = jnp.ones((16,), jnp.float32)
    return ()
  lax.fori_loop(0, _BATCH // 16, _init_o, ())

  prev_rds = None
  for p in range(2):
    q = 2 * p + c
    lo = q * _QROWS

    def _zero(k):
      return pltpu.async_copy(
          zb, hist.at[pl.ds(s * _SLICE + k * _ZCH, _ZCH)], zsem)

    def _start_loads(b):
      n = _BATCH if b < _NBATCH - 1 else _TAIL
      ebase = s * _EPT + b * _BATCH
      et_v, dst_v, _ = ebufs[b % 2]
      return (
          pltpu.async_copy(et_hbm.at[pl.ds(ebase, n)],
                           et_v.at[pl.ds(0, n)], lsem),
          pltpu.async_copy(dst_hbm.at[pl.ds(ebase, n)],
                           dst_v.at[pl.ds(0, n)], lsem),
      )

    def _group(g, et_v, dst_v, ib):
      off = g * 16
      et = et_v[pl.ds(off, 16)]
      rel = dst_v[pl.ds(off, 16)] - lo
      inq = (rel >= 0) & (rel < _QROWS)
      idx = jnp.where(
          inq, (et >> 7) * _PLANE + rel * _D, _TRASH) + (et & 127)
      ib[pl.ds(off, 16)] = idx

    def _compute(b):
      et_v, dst_v, ib = ebufs[b % 2]
      if b < _NBATCH - 1:
        def _rows(r, _):
          for c8 in range(8):
            _group(r * 8 + c8, et_v, dst_v, ib)
          return ()
        lax.fori_loop(0, 16, _rows, ())
      else:
        def _rows_t(r, _):
          for c8 in range(8):
            _group(r * 8 + c8, et_v, dst_v, ib)
          return ()
        lax.fori_loop(0, _TGRP // 8, _rows_t, ())
        for g in range(_TGRP - _TGRP % 8, _TGRP):
          _group(g, et_v, dst_v, ib)
        tv = _TRASH + lax.iota(jnp.int32, 16)
        def _tf(g, _):
          ib[pl.ds(g * 16, 16)] = tv
          return ()
        lax.fori_loop(_TGRP, _BATCH // 16, _tf, ())

    pend = {0: _start_loads(0), 1: _start_loads(1)}
    if prev_rds is None:
      zds = [_zero(k) for k in range(_SLICE // _ZCH)]
    else:
      zds = []
      for kk in range(_NRCH):
        prev_rds[kk].wait()
        zds.append(_zero(2 * kk))
        zds.append(_zero(2 * kk + 1))
    for d in pend.pop(0):
      d.wait()
    _compute(0)
    for d in zds:
      d.wait()
    plsc.subcore_barrier()

    pending_scatter = [
        pltpu.async_copy(ones_v, hist.at[ebufs[0][2]], ssem, add=True)]
    pend[2] = _start_loads(2)
    for b in range(1, _NBATCH):
      et_v, dst_v, ib = ebufs[b % 2]
      for d in pend.pop(b):
        d.wait()
      _compute(b)
      for d in pending_scatter:
        d.wait()
      pending_scatter = [
          pltpu.async_copy(ones_v, hist.at[ib], ssem, add=True)]
      if b + 2 < _NBATCH:
        pend[b + 2] = _start_loads(b + 2)
    for d in pending_scatter:
      d.wait()
    plsc.subcore_barrier()

    kpl = s // 4
    dbase = kpl * (_N * _D) + q * _PLANE + (s % 4) * _SLICE
    prev_rds = [
        pltpu.async_copy(
            hist.at[pl.ds(s * _SLICE + kk * _RCH, _RCH)],
            out_hbm.at[pl.ds(dbase + kk * _RCH, _RCH)], rsem)
        for kk in range(_NRCH)
    ]
  for d in prev_rds:
    d.wait()


_sc_hist = functools.partial(
    pl.kernel,
    out_type=jax.ShapeDtypeStruct((_NQ * _NHQ,), jnp.float32),
    mesh=plsc.VectorSubcoreMesh(
        core_axis_name="c", subcore_axis_name="s",
        num_cores=_NC, num_subcores=_NS),
    scratch_types=[
        pltpu.VMEM_SHARED((_HBUF,), jnp.float32),
        pltpu.VMEM((_BATCH,), jnp.int32),
        pltpu.VMEM((_BATCH,), jnp.int32),
        pltpu.VMEM((_BATCH,), jnp.float32),
        pltpu.VMEM((_ZCH,), jnp.float32),
        pltpu.VMEM((_BATCH,), jnp.int32),
        pltpu.VMEM((_BATCH,), jnp.int32),
        pltpu.VMEM((_BATCH,), jnp.int32),
        pltpu.VMEM((_BATCH,), jnp.int32),
        pltpu.SemaphoreType.DMA,
        pltpu.SemaphoreType.DMA,
        pltpu.SemaphoreType.DMA,
        pltpu.SemaphoreType.DMA,
    ],
)(_sc_hist_body)


def _mm_body(h_ref, t_ref, o_ref):
  acc = jnp.zeros((h_ref.shape[1], _D), jnp.float32)
  deg = jnp.zeros((h_ref.shape[1], 1), jnp.float32)
  for k in range(_KP):
    hk = h_ref[k]
    acc += jnp.dot(hk, t_ref[k], preferred_element_type=jnp.float32)
    deg += jnp.sum(hk, axis=1, keepdims=True)
  o_ref[...] = acc / jnp.maximum(deg, 1.0)


def _tc_combine(hist3, table3):
  blk = 2000
  return pl.pallas_call(
      _mm_body,
      grid=(_N // blk,),
      in_specs=[
          pl.BlockSpec((_KP, blk, _D), lambda i: (0, i, 0)),
          pl.BlockSpec((_KP, _D, _D), lambda i: (0, 0, 0)),
      ],
      out_specs=pl.BlockSpec((blk, _D), lambda i: (i, 0)),
      out_shape=jax.ShapeDtypeStruct((_N, _D), jnp.float32),
  )(hist3, table3)


@jax.jit
def kernel(rel_head_emb, rel_tail_emb, etypes, dst):
  table = jnp.zeros((_TPAD, _D), jnp.float32)
  table = lax.dynamic_update_slice(table, rel_head_emb, (0, 0))
  table = lax.dynamic_update_slice(table, rel_tail_emb, (_NUM_REL, 0))
  table3 = table.reshape(_KP, _D, _D)

  hist3 = _sc_hist(etypes, dst).reshape(_KP, _N, _D)
  return _tc_combine(hist3, table3)
